# baseline (speedup 1.0000x reference)
import functools

import jax
import jax.numpy as jnp
from jax import lax
from jax.experimental import pallas as pl
from jax.experimental.pallas import tpu as pltpu
from jax.experimental.pallas import tpu_sc as plsc

_CHUNK = 128


@functools.cache
def _build(num_emb, dim, batch):
    mesh = plsc.VectorSubcoreMesh(core_axis_name="c", subcore_axis_name="s")
    nc, ns = mesh.num_cores, mesh.num_subcores
    nw = nc * ns
    b_per_w = batch // nw
    n_chunks = b_per_w // _CHUNK

    @functools.partial(
        pl.kernel,
        mesh=mesh,
        out_type=jax.ShapeDtypeStruct((batch, dim), jnp.float32),
        scratch_types=[
            pltpu.VMEM((n_chunks, _CHUNK), jnp.int32),
            pltpu.VMEM((b_per_w, dim), jnp.float32),
            pltpu.SemaphoreType.DMA,
        ],
        compiler_params=pltpu.CompilerParams(use_tc_tiling_on_sc=False),
    )
    def gather_kernel(idx_hbm, table_hbm, out_hbm, idx_v, rows_v, sem):
        wid = lax.axis_index("s") * nc + lax.axis_index("c")
        base = wid * b_per_w
        pltpu.sync_copy(idx_hbm.at[wid], idx_v)
        for j in range(n_chunks):
            pltpu.async_copy(
                table_hbm.at[idx_v.at[j]],
                rows_v.at[pl.ds(j * _CHUNK, _CHUNK)],
                sem,
            )
        for j in range(n_chunks):
            pltpu.make_async_copy(
                table_hbm.at[idx_v.at[j]],
                rows_v.at[pl.ds(j * _CHUNK, _CHUNK)],
                sem,
            ).wait()
        pltpu.sync_copy(rows_v, out_hbm.at[pl.ds(base, b_per_w)])

    def run(idx, table):
        idx3 = idx.reshape(nw, n_chunks, _CHUNK)
        return gather_kernel(idx3, table)

    return run


def kernel(input_exs, table):
    idx = jnp.squeeze(input_exs, axis=1).astype(jnp.int32)
    run = _build(table.shape[0], table.shape[1], idx.shape[0])
    return run(idx, table)


# speedup vs baseline: 2.3666x; 2.3666x over previous
import functools

import jax
import jax.numpy as jnp
from jax import lax
from jax.experimental import pallas as pl
from jax.experimental.pallas import tpu as pltpu
from jax.experimental.pallas import tpu_sc as plsc

_LANES = 16
_REC = 128


@functools.cache
def _build(num_emb, dim, batch):
    mesh = plsc.VectorSubcoreMesh(core_axis_name="c", subcore_axis_name="s")
    nc, ns = mesh.num_cores, mesh.num_subcores
    half = batch // nc
    n_tiles = (num_emb + 127) // 128
    tpw = (n_tiles + ns - 1) // ns
    ntl = ((tpw + 2 * _LANES) // _LANES) * _LANES
    n_groups = half // _LANES
    out_block = half // ns

    @functools.partial(
        pl.kernel,
        mesh=mesh,
        out_type=jax.ShapeDtypeStruct((dim, batch), jnp.float32),
        scratch_types=[
            pltpu.VMEM((half,), jnp.int32),
            pltpu.VMEM((half + _LANES,), jnp.int32),
            pltpu.VMEM((half + _LANES,), jnp.int32),
            pltpu.VMEM((half + _LANES,), jnp.int32),
            pltpu.VMEM((half + _LANES,), jnp.int32),
            pltpu.VMEM((ntl,), jnp.int32),
            pltpu.VMEM((ntl,), jnp.int32),
            pltpu.VMEM((ntl,), jnp.int32),
            pltpu.VMEM((ntl,), jnp.int32),
            pltpu.VMEM((2, dim, 128), jnp.float32),
            pltpu.VMEM((_REC, _REC), jnp.float32),
            pltpu.VMEM((_REC,), jnp.int32),
            pltpu.VMEM((_REC, _REC), jnp.float32),
            pltpu.VMEM((dim, 128), jnp.float32),
            pltpu.HBM((batch + _REC, _REC), jnp.float32),
            pltpu.SemaphoreType.DMA,
        ],
        compiler_params=pltpu.CompilerParams(needs_layout_passes=False),
    )
    def gather_kernel(idx_hbm, tab_t_hbm, out_t_hbm, idx_v, hit_s, hit_j,
                      srt_s, srt_j, cnt_v, off_v, run_v, tlist_v, blk_v,
                      loc_v, locjj_v, rec_v, chunk_v, shared_v, fsem):
        sc = lax.axis_index("c")
        u = lax.axis_index("s")
        lanes = lax.iota(jnp.int32, _LANES)
        ones = jnp.ones((_LANES,), jnp.int32)
        zeros = jnp.zeros((_LANES,), jnp.int32)
        lo = u * tpw
        hi = jnp.minimum(lo + tpw, n_tiles)

        pltpu.sync_copy(idx_hbm.at[pl.ds(sc * half, half)], idx_v)

        def reset_dests(_g, carry):
            locjj_v[pl.ds(_g * _LANES, _LANES)] = batch + _g * _LANES + lanes
            return carry

        lax.fori_loop(0, _REC // _LANES, reset_dests, 0)

        def zero_body(g, carry):
            cnt_v[pl.ds(g * _LANES, _LANES)] = zeros
            return carry

        lax.fori_loop(0, ntl // _LANES, zero_body, 0)

        def scan_body(g, nh):
            s = idx_v[pl.ds(g * _LANES, _LANES)]
            t = lax.shift_right_logical(s, 7)
            m = (t >= lo) & (t < hi)
            pc = plsc.all_reduce_population_count(m)[0]
            plsc.store_compressed(hit_s.at[pl.ds(nh, _LANES)], s, mask=m)
            plsc.store_compressed(hit_j.at[pl.ds(nh, _LANES)],
                                  g * _LANES + lanes, mask=m)
            tc = jnp.clip(t - lo, 0, ntl - 1)
            plsc.addupdate_scatter(cnt_v, [tc], ones, mask=m)
            return nh + pc

        nhits = lax.fori_loop(0, n_groups, scan_body, 0)
        nhg = (nhits + _LANES - 1) // _LANES

        def prefix_body(g, carry):
            base, nm = carry
            c = cnt_v[pl.ds(g * _LANES, _LANES)]
            inc = plsc.cumsum(c)
            excl = base + inc - c
            off_v[pl.ds(g * _LANES, _LANES)] = excl
            run_v[pl.ds(g * _LANES, _LANES)] = excl
            m = c > 0
            pc = plsc.all_reduce_population_count(m)[0]
            plsc.store_compressed(tlist_v.at[pl.ds(nm, _LANES)],
                                  g * _LANES + lanes, mask=m)
            return base + inc[_LANES - 1], nm + pc

        _, nmarked = lax.fori_loop(0, ntl // _LANES, prefix_body, (0, 0))

        lane0 = lanes == 0

        def sort_body(g, carry):
            s = hit_s[pl.ds(g * _LANES, _LANES)]
            jjv = hit_j[pl.ds(g * _LANES, _LANES)]
            for l in range(_LANES):
                @pl.when(g * _LANES + l < nhits)
                def _do(l=l):
                    s_l = s[l]
                    j_l = jjv[l]
                    t_l = lax.shift_right_logical(s_l, 7) - lo
                    pv = run_v[pl.ds(t_l, _LANES)]
                    pos = pv[0]
                    plsc.store_scatter(
                        srt_s, [jnp.full((_LANES,), pos, jnp.int32)],
                        jnp.full((_LANES,), s_l, jnp.int32), mask=lane0)
                    plsc.store_scatter(
                        srt_j, [jnp.full((_LANES,), pos, jnp.int32)],
                        jnp.full((_LANES,), j_l, jnp.int32), mask=lane0)
                    plsc.addupdate_scatter(
                        run_v, [jnp.full((_LANES,), t_l, jnp.int32)], ones,
                        mask=lane0)
            return carry

        lax.fori_loop(0, nhg, sort_body, 0)

        def flush():
            pltpu.sync_copy(loc_v, shared_v.at[locjj_v])
            lax.fori_loop(0, _REC // _LANES, reset_dests, 0)

        def fetch(i):
            tv = tlist_v[pl.ds(i, _LANES)]
            t = tv[0] + lo
            woff = pl.multiple_of(t * 128, 128)
            pltpu.async_copy(
                tab_t_hbm.at[:, pl.ds(woff, 128)],
                blk_v.at[lax.rem(i, 2)],
                fsem,
            )

        def tile_body(i, nd):
            @pl.when(i == 0)
            def _prime():
                fetch(0)

            @pl.when(i + 1 < nmarked)
            def _prefetch():
                fetch(i + 1)

            pltpu.make_async_copy(
                tab_t_hbm.at[:, pl.ds(0, 128)], blk_v.at[0], fsem
            ).wait()

            tv = tlist_v[pl.ds(i, _LANES)]
            tl = tv[0]
            ov = off_v[pl.ds(tl, _LANES)]
            o0 = ov[0]
            cv = cnt_v[pl.ds(tl, _LANES)]
            cn = cv[0]
            bufs = jnp.full((_LANES,), lax.rem(i, 2), jnp.int32)

            def ext_body(e, nd2):
                sv = srt_s[pl.ds(o0 + e, _LANES)]
                s = sv[0]
                jv = srt_j[pl.ds(o0 + e, _LANES)]
                jj = jv[0]
                c = lax.rem(s, 128)
                cs = jnp.full((_LANES,), c, jnp.int32)
                slot = lax.rem(nd2, _REC)
                slots = jnp.full((_LANES,), slot, jnp.int32)
                for h in range(dim // _LANES):
                    d_idx = lanes + _LANES * h
                    vals = plsc.load_gather(blk_v, [bufs, d_idx, cs])
                    plsc.store_scatter(loc_v, [slots, d_idx], vals)
                plsc.store_scatter(locjj_v, [slots],
                                   jnp.full((_LANES,), sc * half + jj, jnp.int32))

                @pl.when(slot == _REC - 1)
                def _full():
                    flush()

                return nd2 + 1

            return lax.fori_loop(0, cn, ext_body, nd)

        ndone = lax.fori_loop(0, nmarked, tile_body, 0)

        @pl.when(lax.rem(ndone, _REC) != 0)
        def _tail():
            flush()

        plsc.subcore_barrier()

        for q in range(out_block // _REC):
            pltpu.sync_copy(
                shared_v.at[pl.ds(sc * half + u * out_block + q * _REC, _REC)], rec_v)

            def tr_body(d, carry):
                ds_ = jnp.full((_LANES,), d, jnp.int32)
                for cg in range(_REC // _LANES):
                    cvec = cg * _LANES + lanes
                    vals = plsc.load_gather(rec_v, [cvec, ds_])
                    plsc.store_scatter(chunk_v, [ds_, cvec], vals)
                return carry

            lax.fori_loop(0, dim, tr_body, 0)
            pltpu.sync_copy(
                chunk_v,
                out_t_hbm.at[:, pl.ds(sc * half + u * out_block + q * _REC,
                                      _REC)],
            )

    return gather_kernel


def kernel(input_exs, table):
    idx = input_exs.reshape(-1).astype(jnp.int32)
    run = _build(table.shape[0], table.shape[1], idx.shape[0])
    out_t = run(idx, table.T)
    return out_t.T


# speedup vs baseline: 2.8686x; 1.2121x over previous
import functools

import jax
import jax.numpy as jnp
from jax import lax
from jax.experimental import pallas as pl
from jax.experimental.pallas import tpu as pltpu
from jax.experimental.pallas import tpu_sc as plsc

_LANES = 16
_REC = 128


@functools.cache
def _build(num_emb, dim, batch):
    mesh = plsc.VectorSubcoreMesh(core_axis_name="c", subcore_axis_name="s")
    nc, ns = mesh.num_cores, mesh.num_subcores
    nw = nc * ns
    n_tiles = (num_emb + 127) // 128
    tpw = (n_tiles + nw - 1) // nw
    ntl = ((tpw + 2 * _LANES) // _LANES) * _LANES
    n_groups = batch // _LANES
    out_block = batch // nw

    @functools.partial(
        pl.kernel,
        mesh=mesh,
        out_type=jax.ShapeDtypeStruct((batch + _REC, _REC), jnp.float32),
        scratch_types=[
            pltpu.VMEM((batch,), jnp.int32),
            pltpu.VMEM((batch + _LANES,), jnp.int32),
            pltpu.VMEM((batch + _LANES,), jnp.int32),
            pltpu.VMEM((ntl,), jnp.int32),
            pltpu.VMEM((ntl,), jnp.int32),
            pltpu.VMEM((ntl,), jnp.int32),
            pltpu.VMEM((ntl,), jnp.int32),
            pltpu.VMEM((2, dim, 128), jnp.float32),
            pltpu.VMEM((_REC, _REC), jnp.float32),
            pltpu.VMEM((_REC,), jnp.int32),
            pltpu.SemaphoreType.DMA,
        ],
        compiler_params=pltpu.CompilerParams(needs_layout_passes=False),
    )
    def produce(idx_hbm, tab_t_hbm, rec_hbm, idx_v, hit_v, srt_v, cnt_v,
                off_v, run_v, tlist_v, blk_v, loc_v, locjj_v, fsem):
        sc = lax.axis_index("c")
        u = lax.axis_index("s")
        w = u * nc + sc
        lanes = lax.iota(jnp.int32, _LANES)
        ones = jnp.ones((_LANES,), jnp.int32)
        zeros = jnp.zeros((_LANES,), jnp.int32)
        lo = w * tpw
        hi = jnp.minimum(lo + tpw, n_tiles)

        pltpu.sync_copy(idx_hbm, idx_v)

        def reset_dests(_g, carry):
            locjj_v[pl.ds(_g * _LANES, _LANES)] = batch + _g * _LANES + lanes
            return carry

        lax.fori_loop(0, _REC // _LANES, reset_dests, 0)

        def zero_body(g, carry):
            cnt_v[pl.ds(g * _LANES, _LANES)] = zeros
            return carry

        lax.fori_loop(0, ntl // _LANES, zero_body, 0)

        def scan_body(g, nh):
            s = idx_v[pl.ds(g * _LANES, _LANES)]
            t = lax.shift_right_logical(s, 7)
            m = (t >= lo) & (t < hi)
            pc = plsc.all_reduce_population_count(m)[0]
            tl = jnp.clip(t - lo, 0, ntl - 1)
            rec = (tl << 21) | ((g * _LANES + lanes) << 7) | (s & 127)
            plsc.store_compressed(hit_v.at[pl.ds(nh, _LANES)], rec, mask=m)
            plsc.addupdate_scatter(cnt_v, [tl], ones, mask=m)
            return nh + pc

        nhits = lax.fori_loop(0, n_groups, scan_body, 0)
        nhg = (nhits + _LANES - 1) // _LANES

        def prefix_body(g, carry):
            base, nm = carry
            c = cnt_v[pl.ds(g * _LANES, _LANES)]
            inc = plsc.cumsum(c)
            excl = base + inc - c
            off_v[pl.ds(g * _LANES, _LANES)] = excl
            run_v[pl.ds(g * _LANES, _LANES)] = excl
            m = c > 0
            pc = plsc.all_reduce_population_count(m)[0]
            plsc.store_compressed(tlist_v.at[pl.ds(nm, _LANES)],
                                  g * _LANES + lanes, mask=m)
            return base + inc[_LANES - 1], nm + pc

        _, nmarked = lax.fori_loop(0, ntl // _LANES, prefix_body, (0, 0))

        lane0 = lanes == 0

        def sort_body(g, carry):
            r = hit_v[pl.ds(g * _LANES, _LANES)]
            for l in range(_LANES):
                @pl.when(g * _LANES + l < nhits)
                def _do(l=l):
                    r_l = r[l]
                    t_l = lax.shift_right_logical(r_l, 21)
                    pv = run_v[pl.ds(t_l, _LANES)]
                    pos = pv[0]
                    plsc.store_scatter(
                        srt_v, [jnp.full((_LANES,), pos, jnp.int32)],
                        jnp.full((_LANES,), r_l, jnp.int32), mask=lane0)
                    plsc.addupdate_scatter(
                        run_v, [jnp.full((_LANES,), t_l, jnp.int32)], ones,
                        mask=lane0)
            return carry

        lax.fori_loop(0, nhg, sort_body, 0)

        def flush():
            pltpu.sync_copy(loc_v, rec_hbm.at[locjj_v])
            lax.fori_loop(0, _REC // _LANES, reset_dests, 0)

        def fetch(i):
            tv = tlist_v[pl.ds(i, _LANES)]
            t = tv[0] + lo
            woff = pl.multiple_of(t * 128, 128)
            pltpu.async_copy(
                tab_t_hbm.at[:, pl.ds(woff, 128)],
                blk_v.at[lax.rem(i, 2)],
                fsem,
            )

        def tile_body(i, nd):
            @pl.when(i == 0)
            def _prime():
                fetch(0)

            @pl.when(i + 1 < nmarked)
            def _prefetch():
                fetch(i + 1)

            pltpu.make_async_copy(
                tab_t_hbm.at[:, pl.ds(0, 128)], blk_v.at[0], fsem
            ).wait()

            tv = tlist_v[pl.ds(i, _LANES)]
            tl = tv[0]
            ov = off_v[pl.ds(tl, _LANES)]
            o0 = ov[0]
            cv = cnt_v[pl.ds(tl, _LANES)]
            cn = cv[0]
            bufs = jnp.full((_LANES,), lax.rem(i, 2), jnp.int32)

            def ext_body(e, nd2):
                rv = srt_v[pl.ds(o0 + e, _LANES)]
                r = rv[0]
                jj = lax.shift_right_logical(r, 7) & (batch - 1)
                c = r & 127
                cs = jnp.full((_LANES,), c, jnp.int32)
                slot = lax.rem(nd2, _REC)
                slots = jnp.full((_LANES,), slot, jnp.int32)
                for h in range(dim // _LANES):
                    d_idx = lanes + _LANES * h
                    vals = plsc.load_gather(blk_v, [bufs, d_idx, cs])
                    plsc.store_scatter(loc_v, [slots, d_idx], vals)
                plsc.store_scatter(locjj_v, [slots],
                                   jnp.full((_LANES,), jj, jnp.int32))

                @pl.when(slot == _REC - 1)
                def _full():
                    flush()

                return nd2 + 1

            return lax.fori_loop(0, cn, ext_body, nd)

        ndone = lax.fori_loop(0, nmarked, tile_body, 0)

        @pl.when(lax.rem(ndone, _REC) != 0)
        def _tail():
            flush()

    @functools.partial(
        pl.kernel,
        mesh=mesh,
        out_type=jax.ShapeDtypeStruct((dim, batch), jnp.float32),
        scratch_types=[
            pltpu.VMEM((_REC, _REC), jnp.float32),
            pltpu.VMEM((dim, 128), jnp.float32),
        ],
        compiler_params=pltpu.CompilerParams(needs_layout_passes=False),
    )
    def consume(rec_hbm, out_t_hbm, rec_v, chunk_v):
        sc = lax.axis_index("c")
        u = lax.axis_index("s")
        w = u * nc + sc
        base = w * out_block
        lanes = lax.iota(jnp.int32, _LANES)

        for q in range(out_block // _REC):
            pltpu.sync_copy(rec_hbm.at[pl.ds(base + q * _REC, _REC)], rec_v)

            def tr_body(d, carry):
                ds_ = jnp.full((_LANES,), d, jnp.int32)
                for cg in range(_REC // _LANES):
                    cvec = cg * _LANES + lanes
                    vals = plsc.load_gather(rec_v, [cvec, ds_])
                    plsc.store_scatter(chunk_v, [ds_, cvec], vals)
                return carry

            lax.fori_loop(0, dim, tr_body, 0)
            pltpu.sync_copy(
                chunk_v, out_t_hbm.at[:, pl.ds(base + q * _REC, _REC)])

    def run(idx, tab_t):
        rec = produce(idx, tab_t)
        return consume(rec)

    return run


def kernel(input_exs, table):
    idx = input_exs.reshape(-1).astype(jnp.int32)
    run = _build(table.shape[0], table.shape[1], idx.shape[0])
    out_t = run(idx, table.T)
    return out_t.T


# speedup vs baseline: 3.1938x; 1.1134x over previous
import functools

import jax
import jax.numpy as jnp
from jax import lax
from jax.experimental import pallas as pl
from jax.experimental.pallas import tpu as pltpu
from jax.experimental.pallas import tpu_sc as plsc

_LANES = 16
_REC = 128


@functools.cache
def _build(num_emb, dim, batch):
    mesh = plsc.VectorSubcoreMesh(core_axis_name="c", subcore_axis_name="s")
    nc, ns = mesh.num_cores, mesh.num_subcores
    nw = nc * ns
    n_tiles = (num_emb + 127) // 128
    tpw = (n_tiles + nw - 1) // nw
    ntl = ((tpw + 2 * _LANES) // _LANES) * _LANES
    n_groups = batch // _LANES
    out_block = batch // nw

    @functools.partial(
        pl.kernel,
        mesh=mesh,
        out_type=jax.ShapeDtypeStruct((batch + _REC, _REC), jnp.float32),
        scratch_types=[
            pltpu.VMEM((batch,), jnp.int32),
            pltpu.VMEM((batch + _LANES,), jnp.int32),
            pltpu.VMEM((batch + _LANES,), jnp.int32),
            pltpu.VMEM((ntl,), jnp.int32),
            pltpu.VMEM((ntl,), jnp.int32),
            pltpu.VMEM((ntl,), jnp.int32),
            pltpu.VMEM((ntl,), jnp.int32),
            pltpu.VMEM((2, dim, 128), jnp.float32),
            pltpu.VMEM((_REC, _REC), jnp.float32),
            pltpu.VMEM((_REC,), jnp.int32),
            pltpu.SemaphoreType.DMA,
        ],
        compiler_params=pltpu.CompilerParams(needs_layout_passes=False),
    )
    def produce(idx_hbm, tab_t_hbm, rec_hbm, idx_v, hit_v, srt_v, cnt_v,
                off_v, run_v, tlist_v, blk_v, loc_v, locjj_v, fsem):
        sc = lax.axis_index("c")
        u = lax.axis_index("s")
        w = u * nc + sc
        lanes = lax.iota(jnp.int32, _LANES)
        ones = jnp.ones((_LANES,), jnp.int32)
        zeros = jnp.zeros((_LANES,), jnp.int32)
        lo = w * tpw
        hi = jnp.minimum(lo + tpw, n_tiles)

        pltpu.sync_copy(idx_hbm, idx_v)

        def reset_dests(_g, carry):
            locjj_v[pl.ds(_g * _LANES, _LANES)] = batch + _g * _LANES + lanes
            return carry

        lax.fori_loop(0, _REC // _LANES, reset_dests, 0)

        def zero_body(g, carry):
            cnt_v[pl.ds(g * _LANES, _LANES)] = zeros
            return carry

        lax.fori_loop(0, ntl // _LANES, zero_body, 0)

        def scan_body(g, nh):
            s = idx_v[pl.ds(g * _LANES, _LANES)]
            t = lax.shift_right_logical(s, 7)
            m = (t >= lo) & (t < hi)
            pc = plsc.all_reduce_population_count(m)[0]
            tl = jnp.clip(t - lo, 0, ntl - 1)
            rec = (tl << 21) | ((g * _LANES + lanes) << 7) | (s & 127)
            plsc.store_compressed(hit_v.at[pl.ds(nh, _LANES)], rec, mask=m)
            plsc.addupdate_scatter(cnt_v, [tl], ones, mask=m)
            return nh + pc

        nhits = lax.fori_loop(0, n_groups, scan_body, 0)
        nhg = (nhits + _LANES - 1) // _LANES

        def prefix_body(g, carry):
            base, nm = carry
            c = cnt_v[pl.ds(g * _LANES, _LANES)]
            inc = plsc.cumsum(c)
            excl = base + inc - c
            off_v[pl.ds(g * _LANES, _LANES)] = excl
            run_v[pl.ds(g * _LANES, _LANES)] = excl
            m = c > 0
            pc = plsc.all_reduce_population_count(m)[0]
            plsc.store_compressed(tlist_v.at[pl.ds(nm, _LANES)],
                                  g * _LANES + lanes, mask=m)
            return base + inc[_LANES - 1], nm + pc

        _, nmarked = lax.fori_loop(0, ntl // _LANES, prefix_body, (0, 0))

        lane0 = lanes == 0

        def sort_body(g, carry):
            r = hit_v[pl.ds(g * _LANES, _LANES)]
            for l in range(_LANES):
                @pl.when(g * _LANES + l < nhits)
                def _do(l=l):
                    r_l = r[l]
                    t_l = lax.shift_right_logical(r_l, 21)
                    pv = run_v[pl.ds(t_l, _LANES)]
                    pos = pv[0]
                    plsc.store_scatter(
                        srt_v, [jnp.full((_LANES,), pos, jnp.int32)],
                        jnp.full((_LANES,), r_l, jnp.int32), mask=lane0)
                    plsc.addupdate_scatter(
                        run_v, [jnp.full((_LANES,), t_l, jnp.int32)], ones,
                        mask=lane0)
            return carry

        lax.fori_loop(0, nhg, sort_body, 0)

        def flush():
            pltpu.sync_copy(loc_v, rec_hbm.at[locjj_v])
            lax.fori_loop(0, _REC // _LANES, reset_dests, 0)

        def fetch(i):
            tv = tlist_v[pl.ds(i, _LANES)]
            t = tv[0] + lo
            woff = pl.multiple_of(t * 128, 128)
            pltpu.async_copy(
                tab_t_hbm.at[:, pl.ds(woff, 128)],
                blk_v.at[lax.rem(i, 2)],
                fsem,
            )

        def tile_body(i, nd):
            @pl.when(i == 0)
            def _prime():
                fetch(0)

            @pl.when(i + 1 < nmarked)
            def _prefetch():
                fetch(i + 1)

            pltpu.make_async_copy(
                tab_t_hbm.at[:, pl.ds(0, 128)], blk_v.at[0], fsem
            ).wait()

            tv = tlist_v[pl.ds(i, _LANES)]
            tl = tv[0]
            ov = off_v[pl.ds(tl, _LANES)]
            o0 = ov[0]
            cv = cnt_v[pl.ds(tl, _LANES)]
            cn = cv[0]
            bufs = jnp.full((_LANES,), lax.rem(i, 2), jnp.int32)

            def ext_body(e, nd2):
                rv = srt_v[pl.ds(o0 + e, _LANES)]
                r = rv[0]
                jj = lax.shift_right_logical(r, 7) & (batch - 1)
                c = r & 127
                cs = jnp.full((_LANES,), c, jnp.int32)
                slot = lax.rem(nd2, _REC)
                slots = jnp.full((_LANES,), slot, jnp.int32)
                for h in range(dim // _LANES):
                    d_idx = lanes + _LANES * h
                    vals = plsc.load_gather(blk_v, [bufs, d_idx, cs])
                    plsc.store_scatter(loc_v, [slots, d_idx], vals)
                plsc.store_scatter(locjj_v, [slots],
                                   jnp.full((_LANES,), jj, jnp.int32))

                @pl.when(slot == _REC - 1)
                def _full():
                    flush()

                return nd2 + 1

            return lax.fori_loop(0, cn, ext_body, nd)

        ndone = lax.fori_loop(0, nmarked, tile_body, 0)

        @pl.when(lax.rem(ndone, _REC) != 0)
        def _tail():
            flush()

    @functools.partial(
        pl.kernel,
        mesh=mesh,
        out_type=jax.ShapeDtypeStruct((dim, batch), jnp.float32),
        scratch_types=[
            pltpu.VMEM((_REC, _REC), jnp.float32),
            pltpu.VMEM((dim, 128), jnp.float32),
        ],
        compiler_params=pltpu.CompilerParams(needs_layout_passes=False),
    )
    def consume(rec_hbm, out_t_hbm, rec_v, chunk_v):
        sc = lax.axis_index("c")
        u = lax.axis_index("s")
        w = u * nc + sc
        base = w * out_block
        lanes = lax.iota(jnp.int32, _LANES)

        for q in range(out_block // _REC):
            pltpu.sync_copy(rec_hbm.at[pl.ds(base + q * _REC, _REC)], rec_v)

            def tr_body(d, carry):
                ds_ = jnp.full((_LANES,), d, jnp.int32)
                for cg in range(_REC // _LANES):
                    cvec = cg * _LANES + lanes
                    vals = plsc.load_gather(rec_v, [cvec, ds_])
                    plsc.store_scatter(chunk_v, [ds_, cvec], vals)
                return carry

            lax.fori_loop(0, dim, tr_body, 0)
            pltpu.sync_copy(
                chunk_v, out_t_hbm.at[:, pl.ds(base + q * _REC, _REC)])

    def run(idx, tab_t):
        rec = produce(idx, tab_t)
        return rec[:batch, :dim]

    return run


def kernel(input_exs, table):
    idx = input_exs.reshape(-1).astype(jnp.int32)
    run = _build(table.shape[0], table.shape[1], idx.shape[0])
    return run(idx, table.T)


# speedup vs baseline: 4.6817x; 1.4659x over previous
import functools

import jax
import jax.numpy as jnp
from jax import lax
from jax.experimental import pallas as pl
from jax.experimental.pallas import tpu as pltpu
from jax.experimental.pallas import tpu_sc as plsc

_LANES = 16
_REC = 128


@functools.cache
def _build(num_emb, dim, batch):
    mesh = plsc.VectorSubcoreMesh(core_axis_name="c", subcore_axis_name="s")
    nc, ns = mesh.num_cores, mesh.num_subcores
    nw = nc * ns
    n_tiles = (num_emb + 127) // 128
    tpw = (n_tiles + nw - 1) // nw
    ntl = ((tpw + 2 * _LANES) // _LANES) * _LANES
    n_groups = batch // _LANES
    out_block = batch // nw

    @functools.partial(
        pl.kernel,
        mesh=mesh,
        out_type=jax.ShapeDtypeStruct((batch + _REC, _REC), jnp.float32),
        scratch_types=[
            pltpu.VMEM((batch,), jnp.int32),
            pltpu.VMEM((batch + 2 * _LANES,), jnp.int32),
            pltpu.VMEM((batch + _LANES,), jnp.int32),
            pltpu.VMEM((ntl,), jnp.int32),
            pltpu.VMEM((ntl,), jnp.int32),
            pltpu.VMEM((ntl,), jnp.int32),
            pltpu.VMEM((ntl,), jnp.int32),
            pltpu.VMEM((4, dim, 128), jnp.float32),
            pltpu.VMEM((_REC, _REC), jnp.float32),
            pltpu.VMEM((_REC,), jnp.int32),
            pltpu.SemaphoreType.DMA,
        ],
        compiler_params=pltpu.CompilerParams(needs_layout_passes=False),
    )
    def produce(idx_hbm, tab_t_hbm, rec_hbm, idx_v, hit_v, srt_v, cnt_v,
                off_v, run_v, tlist_v, blk_v, loc_v, locjj_v, fsem):
        sc = lax.axis_index("c")
        u = lax.axis_index("s")
        w = u * nc + sc
        lanes = lax.iota(jnp.int32, _LANES)
        ones = jnp.ones((_LANES,), jnp.int32)
        zeros = jnp.zeros((_LANES,), jnp.int32)
        lo = w * tpw
        hi = jnp.minimum(lo + tpw, n_tiles)

        pltpu.sync_copy(idx_hbm, idx_v)

        def reset_dests(_g, carry):
            locjj_v[pl.ds(_g * _LANES, _LANES)] = batch + _g * _LANES + lanes
            return carry

        lax.fori_loop(0, _REC // _LANES, reset_dests, 0)

        def zero_body(g, carry):
            cnt_v[pl.ds(g * _LANES, _LANES)] = zeros
            return carry

        lax.fori_loop(0, ntl // _LANES, zero_body, 0)

        def scan_body(g, nh):
            s = idx_v[pl.ds(g * _LANES, _LANES)]
            t = lax.shift_right_logical(s, 7)
            m = (t >= lo) & (t < hi)
            pc = plsc.all_reduce_population_count(m)[0]
            tl = jnp.clip(t - lo, 0, ntl - 1)
            rec = (tl << 21) | ((g * _LANES + lanes) << 7) | (s & 127)
            plsc.store_compressed(hit_v.at[pl.ds(nh + _LANES, _LANES)], rec,
                                  mask=m)
            plsc.addupdate_scatter(cnt_v, [tl], ones, mask=m)
            return nh + pc

        nhits = lax.fori_loop(0, n_groups, scan_body, 0)
        nhg = (nhits + _LANES - 1) // _LANES

        def prefix_body(g, carry):
            base, nm = carry
            c = cnt_v[pl.ds(g * _LANES, _LANES)]
            inc = plsc.cumsum(c)
            excl = base + inc - c
            off_v[pl.ds(g * _LANES, _LANES)] = excl
            run_v[pl.ds(g * _LANES, _LANES)] = excl
            m = c > 0
            pc = plsc.all_reduce_population_count(m)[0]
            plsc.store_compressed(tlist_v.at[pl.ds(nm, _LANES)],
                                  g * _LANES + lanes, mask=m)
            return base + inc[_LANES - 1], nm + pc

        _, nmarked = lax.fori_loop(0, ntl // _LANES, prefix_body, (0, 0))

        def sort_body(g, carry):
            r = hit_v[pl.ds(_LANES + g * _LANES, _LANES)]
            t = lax.shift_right_logical(r, 21)
            valid = (g * _LANES + lanes) < nhits
            rank = zeros
            for sh in range(1, _LANES):
                r_sh = hit_v[pl.ds(_LANES + g * _LANES - sh, _LANES)]
                t_sh = lax.shift_right_logical(r_sh, 21)
                rank = rank + jnp.where((t == t_sh) & (lanes >= sh), 1, 0)
            tc = jnp.clip(t, 0, ntl - 1)
            pos = plsc.load_gather(run_v, [tc], mask=valid) + rank
            pos = jnp.clip(pos, 0, batch + _LANES - 1)
            plsc.store_scatter(srt_v, [pos], r, mask=valid)
            plsc.addupdate_scatter(run_v, [tc], ones, mask=valid)
            return carry

        lax.fori_loop(0, nhg, sort_body, 0)

        def flush():
            pltpu.sync_copy(loc_v, rec_hbm.at[locjj_v])
            lax.fori_loop(0, _REC // _LANES, reset_dests, 0)

        def fetch(i):
            tv = tlist_v[pl.ds(i, _LANES)]
            t = tv[0] + lo
            woff = pl.multiple_of(t * 128, 128)
            pltpu.async_copy(
                tab_t_hbm.at[:, pl.ds(woff, 128)],
                blk_v.at[lax.rem(i, 4)],
                fsem,
            )

        for j in range(3):
            @pl.when(j < nmarked)
            def _prime(j=j):
                fetch(j)

        def tile_body(i, nd):
            @pl.when(i + 3 < nmarked)
            def _prefetch():
                fetch(i + 3)

            pltpu.make_async_copy(
                tab_t_hbm.at[:, pl.ds(0, 128)], blk_v.at[0], fsem
            ).wait()

            tv = tlist_v[pl.ds(i, _LANES)]
            tl = tv[0]
            ov = off_v[pl.ds(tl, _LANES)]
            o0 = ov[0]
            cv = cnt_v[pl.ds(tl, _LANES)]
            cn = cv[0]
            bufs = jnp.full((_LANES,), lax.rem(i, 4), jnp.int32)

            def ext_body(e, nd2):
                rv = srt_v[pl.ds(o0 + e, _LANES)]
                r = rv[0]
                jj = lax.shift_right_logical(r, 7) & (batch - 1)
                c = r & 127
                cs = jnp.full((_LANES,), c, jnp.int32)
                slot = lax.rem(nd2, _REC)
                slots = jnp.full((_LANES,), slot, jnp.int32)
                for h in range(dim // _LANES):
                    d_idx = lanes + _LANES * h
                    vals = plsc.load_gather(blk_v, [bufs, d_idx, cs])
                    plsc.store_scatter(loc_v, [slots, d_idx], vals)
                plsc.store_scatter(locjj_v, [slots],
                                   jnp.full((_LANES,), jj, jnp.int32))

                @pl.when(slot == _REC - 1)
                def _full():
                    flush()

                return nd2 + 1

            return lax.fori_loop(0, cn, ext_body, nd)

        ndone = lax.fori_loop(0, nmarked, tile_body, 0)

        @pl.when(lax.rem(ndone, _REC) != 0)
        def _tail():
            flush()

    @functools.partial(
        pl.kernel,
        mesh=mesh,
        out_type=jax.ShapeDtypeStruct((dim, batch), jnp.float32),
        scratch_types=[
            pltpu.VMEM((_REC, _REC), jnp.float32),
            pltpu.VMEM((dim, 128), jnp.float32),
        ],
        compiler_params=pltpu.CompilerParams(needs_layout_passes=False),
    )
    def consume(rec_hbm, out_t_hbm, rec_v, chunk_v):
        sc = lax.axis_index("c")
        u = lax.axis_index("s")
        w = u * nc + sc
        base = w * out_block
        lanes = lax.iota(jnp.int32, _LANES)

        for q in range(out_block // _REC):
            pltpu.sync_copy(rec_hbm.at[pl.ds(base + q * _REC, _REC)], rec_v)

            def tr_body(d, carry):
                ds_ = jnp.full((_LANES,), d, jnp.int32)
                for cg in range(_REC // _LANES):
                    cvec = cg * _LANES + lanes
                    vals = plsc.load_gather(rec_v, [cvec, ds_])
                    plsc.store_scatter(chunk_v, [ds_, cvec], vals)
                return carry

            lax.fori_loop(0, dim, tr_body, 0)
            pltpu.sync_copy(
                chunk_v, out_t_hbm.at[:, pl.ds(base + q * _REC, _REC)])

    def run(idx, tab_t):
        rec = produce(idx, tab_t)
        return rec[:batch, :dim]

    return run


def kernel(input_exs, table):
    idx = input_exs.reshape(-1).astype(jnp.int32)
    run = _build(table.shape[0], table.shape[1], idx.shape[0])
    return run(idx, table.T)


# speedup vs baseline: 4.6830x; 1.0003x over previous
import functools

import jax
import jax.numpy as jnp
from jax import lax
from jax.experimental import pallas as pl
from jax.experimental.pallas import tpu as pltpu
from jax.experimental.pallas import tpu_sc as plsc

_LANES = 16
_REC = 128


@functools.cache
def _build(num_emb, dim, batch):
    mesh = plsc.VectorSubcoreMesh(core_axis_name="c", subcore_axis_name="s")
    nc, ns = mesh.num_cores, mesh.num_subcores
    nw = nc * ns
    n_tiles = (num_emb + 127) // 128
    tpw = (n_tiles + nw - 1) // nw
    ntl = ((tpw + 2 * _LANES) // _LANES) * _LANES
    n_groups = batch // _LANES

    @functools.partial(
        pl.kernel,
        mesh=mesh,
        out_type=jax.ShapeDtypeStruct((batch + _REC, _REC), jnp.float32),
        scratch_types=[
            pltpu.VMEM((batch,), jnp.int32),
            pltpu.VMEM((batch + 2 * _LANES,), jnp.int32),
            pltpu.VMEM((batch + _LANES,), jnp.int32),
            pltpu.VMEM((ntl,), jnp.int32),
            pltpu.VMEM((ntl,), jnp.int32),
            pltpu.VMEM((ntl,), jnp.int32),
            pltpu.VMEM((ntl,), jnp.int32),
            pltpu.VMEM((4, dim, 128), jnp.float32),
            pltpu.VMEM((_REC, _REC), jnp.float32),
            pltpu.VMEM((_REC,), jnp.int32),
            pltpu.SemaphoreType.DMA,
        ],
        compiler_params=pltpu.CompilerParams(needs_layout_passes=False),
    )
    def produce(idx_hbm, tab_t_hbm, rec_hbm, idx_v, hit_v, srt_v, cnt_v,
                off_v, run_v, tlist_v, blk_v, loc_v, locjj_v, fsem):
        sc = lax.axis_index("c")
        u = lax.axis_index("s")
        w = u * nc + sc
        lanes = lax.iota(jnp.int32, _LANES)
        ones = jnp.ones((_LANES,), jnp.int32)
        zeros = jnp.zeros((_LANES,), jnp.int32)
        lo = w * tpw
        hi = jnp.minimum(lo + tpw, n_tiles)

        pltpu.sync_copy(idx_hbm, idx_v)

        def reset_dests(_g, carry):
            locjj_v[pl.ds(_g * _LANES, _LANES)] = batch + _g * _LANES + lanes
            return carry

        lax.fori_loop(0, _REC // _LANES, reset_dests, 0)

        def zero_body(g, carry):
            cnt_v[pl.ds(g * _LANES, _LANES)] = zeros
            return carry

        lax.fori_loop(0, ntl // _LANES, zero_body, 0)

        def scan_body(g, nh):
            s = idx_v[pl.ds(g * _LANES, _LANES)]
            t = lax.shift_right_logical(s, 7)
            m = (t >= lo) & (t < hi)
            pc = plsc.all_reduce_population_count(m)[0]
            tl = jnp.clip(t - lo, 0, ntl - 1)
            rec = (tl << 21) | ((g * _LANES + lanes) << 7) | (s & 127)
            plsc.store_compressed(hit_v.at[pl.ds(nh + _LANES, _LANES)], rec,
                                  mask=m)
            plsc.addupdate_scatter(cnt_v, [tl], ones, mask=m)
            return nh + pc

        nhits = lax.fori_loop(0, n_groups, scan_body, 0)
        nhg = (nhits + _LANES - 1) // _LANES

        def prefix_body(g, carry):
            base, nm = carry
            c = cnt_v[pl.ds(g * _LANES, _LANES)]
            inc = plsc.cumsum(c)
            excl = base + inc - c
            off_v[pl.ds(g * _LANES, _LANES)] = excl
            run_v[pl.ds(g * _LANES, _LANES)] = excl
            m = c > 0
            pc = plsc.all_reduce_population_count(m)[0]
            plsc.store_compressed(tlist_v.at[pl.ds(nm, _LANES)],
                                  g * _LANES + lanes, mask=m)
            return base + inc[_LANES - 1], nm + pc

        _, nmarked = lax.fori_loop(0, ntl // _LANES, prefix_body, (0, 0))

        def sort_body(g, carry):
            r = hit_v[pl.ds(_LANES + g * _LANES, _LANES)]
            t = lax.shift_right_logical(r, 21)
            valid = (g * _LANES + lanes) < nhits
            rank = zeros
            for sh in range(1, _LANES):
                r_sh = hit_v[pl.ds(_LANES + g * _LANES - sh, _LANES)]
                t_sh = lax.shift_right_logical(r_sh, 21)
                rank = rank + jnp.where((t == t_sh) & (lanes >= sh), 1, 0)
            tc = jnp.clip(t, 0, ntl - 1)
            pos = plsc.load_gather(run_v, [tc], mask=valid) + rank
            pos = jnp.clip(pos, 0, batch + _LANES - 1)
            plsc.store_scatter(srt_v, [pos], r, mask=valid)
            plsc.addupdate_scatter(run_v, [tc], ones, mask=valid)
            return carry

        lax.fori_loop(0, nhg, sort_body, 0)

        def flush():
            pltpu.sync_copy(loc_v, rec_hbm.at[locjj_v])
            lax.fori_loop(0, _REC // _LANES, reset_dests, 0)

        def fetch(i):
            tv = tlist_v[pl.ds(i, _LANES)]
            t = tv[0] + lo
            woff = pl.multiple_of(t * 128, 128)
            pltpu.async_copy(
                tab_t_hbm.at[:, pl.ds(woff, 128)],
                blk_v.at[lax.rem(i, 4)],
                fsem,
            )

        for j in range(3):
            @pl.when(j < nmarked)
            def _prime(j=j):
                fetch(j)

        def tile_body(i, nd):
            @pl.when(i + 3 < nmarked)
            def _prefetch():
                fetch(i + 3)

            pltpu.make_async_copy(
                tab_t_hbm.at[:, pl.ds(0, 128)], blk_v.at[0], fsem
            ).wait()

            tv = tlist_v[pl.ds(i, _LANES)]
            tl = tv[0]
            ov = off_v[pl.ds(tl, _LANES)]
            o0 = ov[0]
            cv = cnt_v[pl.ds(tl, _LANES)]
            cn = cv[0]
            bufs = jnp.full((_LANES,), lax.rem(i, 4), jnp.int32)

            def ext_body(e, nd2):
                rv = srt_v[pl.ds(o0 + e, _LANES)]
                r = rv[0]
                jj = lax.shift_right_logical(r, 7) & (batch - 1)
                c = r & 127
                cs = jnp.full((_LANES,), c, jnp.int32)
                slot = lax.rem(nd2, _REC)
                slots = jnp.full((_LANES,), slot, jnp.int32)
                for h in range(dim // _LANES):
                    d_idx = lanes + _LANES * h
                    vals = plsc.load_gather(blk_v, [bufs, d_idx, cs])
                    plsc.store_scatter(loc_v, [slots, d_idx], vals)
                plsc.store_scatter(locjj_v, [slots],
                                   jnp.full((_LANES,), jj, jnp.int32))

                @pl.when(slot == _REC - 1)
                def _full():
                    flush()

                return nd2 + 1

            return lax.fori_loop(0, cn, ext_body, nd)

        ndone = lax.fori_loop(0, nmarked, tile_body, 0)

        @pl.when(lax.rem(ndone, _REC) != 0)
        def _tail():
            flush()

    def run(idx, tab_t):
        rec = produce(idx, tab_t)
        return rec[:batch, :dim]

    return run


def kernel(input_exs, table):
    idx = input_exs.reshape(-1).astype(jnp.int32)
    run = _build(table.shape[0], table.shape[1], idx.shape[0])
    return run(idx, table.T)
